# Initial kernel scaffold; baseline (speedup 1.0000x reference)
import jax
import jax.numpy as jnp
from jax.experimental import pallas as pl


def kernel(tokens, embedding):
    raise NotImplementedError("write your pallas kernel here")


# speedup vs baseline: 3.4294x; 3.4294x over previous
import functools
import math

import jax
import jax.numpy as jnp
from jax import lax
from jax.experimental import pallas as pl
from jax.experimental.pallas import tpu as pltpu
from jax.experimental.pallas import tpu_sc as plsc


def _emb_lookup(idx, table, n_per, chunk, scale):
    N, = idx.shape
    V, D = table.shape
    n_chunks = n_per // chunk
    nc = 2
    mesh = plsc.VectorSubcoreMesh(core_axis_name="c", subcore_axis_name="s")

    @functools.partial(
        pl.kernel,
        mesh=mesh,
        out_type=jax.ShapeDtypeStruct((N, D), jnp.float32),
        scratch_types=[
            pltpu.VMEM((chunk,), jnp.int32),
            pltpu.VMEM((chunk, D), jnp.float32),
            pltpu.SemaphoreType.DMA,
        ],
        compiler_params=pltpu.CompilerParams(use_tc_tiling_on_sc=False),
    )
    def body(idx_hbm, table_hbm, out_hbm, idx_v, rows_v, sem):
        wid = lax.axis_index("s") * nc + lax.axis_index("c")
        base = wid * n_per

        def chunk_body(g, carry):
            start = base + g * chunk
            pltpu.sync_copy(idx_hbm.at[pl.ds(start, chunk)], idx_v)
            pltpu.async_copy(table_hbm.at[idx_v], rows_v, sem).wait()

            def row_body(r, c2):
                for j in range(D // 16):
                    sl = pl.ds(j * 16, 16)
                    rows_v[r, sl] = rows_v[r, sl] * scale
                return c2

            lax.fori_loop(0, chunk, row_body, 0)
            pltpu.sync_copy(rows_v, out_hbm.at[pl.ds(start, chunk)])
            return carry

        lax.fori_loop(0, n_chunks, chunk_body, 0)

    return body(idx, table)


def kernel(tokens, embedding):
    B, S = tokens.shape
    V, D = embedding.shape
    N = B * S
    idx = tokens.reshape(N).astype(jnp.int32)
    n_workers = 32
    n_per = N // n_workers
    out = _emb_lookup(idx, embedding, n_per, 512, math.sqrt(D))
    return out.reshape(B, S, D)


# speedup vs baseline: 4.1861x; 1.2207x over previous
import functools
import math

import jax
import jax.numpy as jnp
from jax import lax
from jax.experimental import pallas as pl
from jax.experimental.pallas import tpu as pltpu
from jax.experimental.pallas import tpu_sc as plsc

_NBUF = 2


def _emb_lookup(idx, table, n_per, chunk, scale):
    N, = idx.shape
    V, D = table.shape
    n_chunks = n_per // chunk
    nc = 2
    mesh = plsc.VectorSubcoreMesh(core_axis_name="c", subcore_axis_name="s")

    @functools.partial(
        pl.kernel,
        mesh=mesh,
        out_type=jax.ShapeDtypeStruct((N, D), jnp.float32),
        scratch_types=[
            pltpu.VMEM((chunk,), jnp.int32),
            pltpu.VMEM((chunk,), jnp.int32),
            pltpu.VMEM((chunk, D), jnp.float32),
            pltpu.VMEM((chunk, D), jnp.float32),
            pltpu.SemaphoreType.DMA((_NBUF,)),
            pltpu.SemaphoreType.DMA((_NBUF,)),
        ],
        compiler_params=pltpu.CompilerParams(use_tc_tiling_on_sc=False),
    )
    def body(idx_hbm, table_hbm, out_hbm, idx_v0, idx_v1, rows_v0, rows_v1,
             gsem, ssem):
        idx_bufs = (idx_v0, idx_v1)
        row_bufs = (rows_v0, rows_v1)
        wid = lax.axis_index("s") * nc + lax.axis_index("c")
        base = wid * n_per

        def start_gather(g, b):
            start = base + g * chunk
            pltpu.sync_copy(idx_hbm.at[pl.ds(start, chunk)], idx_bufs[b])
            pltpu.async_copy(table_hbm.at[idx_bufs[b]], row_bufs[b], gsem.at[b])

        def scale_rows(b):
            def grp_body(r8, carry):
                for r in range(8):
                    for j in range(D // 16):
                        sl = pl.ds(j * 16, 16)
                        row_bufs[b][r8 * 8 + r, sl] = row_bufs[b][r8 * 8 + r, sl] * scale
                return carry

            lax.fori_loop(0, chunk // 8, grp_body, 0)

        def wait_gather(g, b):
            pltpu.make_async_copy(table_hbm.at[idx_bufs[b]], row_bufs[b],
                                  gsem.at[b]).wait()

        def start_scatter(g, b):
            start = base + g * chunk
            pltpu.async_copy(row_bufs[b], out_hbm.at[pl.ds(start, chunk)],
                             ssem.at[b])

        def wait_scatter(g, b):
            start = base + g * chunk
            pltpu.make_async_copy(row_bufs[b],
                                  out_hbm.at[pl.ds(start, chunk)],
                                  ssem.at[b]).wait()

        start_gather(0, 0)

        def pair_body(p, carry):
            for b in range(_NBUF):
                g = p * _NBUF + b
                bn = (b + 1) % _NBUF

                @pl.when(g + 1 < n_chunks)
                def _():
                    @pl.when(g >= 1)
                    def _():
                        wait_scatter(g - 1, bn)

                    start_gather(g + 1, bn)

                wait_gather(g, b)
                scale_rows(b)
                start_scatter(g, b)
            return carry

        lax.fori_loop(0, n_chunks // _NBUF, pair_body, 0)
        wait_scatter(n_chunks - 2, (n_chunks - 2) % _NBUF)
        wait_scatter(n_chunks - 1, (n_chunks - 1) % _NBUF)

    return body(idx, table)


def kernel(tokens, embedding):
    B, S = tokens.shape
    V, D = embedding.shape
    N = B * S
    idx = tokens.reshape(N).astype(jnp.int32)
    n_workers = 32
    n_per = N // n_workers
    out = _emb_lookup(idx, embedding, n_per, 640, math.sqrt(D))
    return out.reshape(B, S, D)


# speedup vs baseline: 4.2592x; 1.0175x over previous
import functools
import math

import jax
import jax.numpy as jnp
from jax import lax
from jax.experimental import pallas as pl
from jax.experimental.pallas import tpu as pltpu
from jax.experimental.pallas import tpu_sc as plsc

_NBUF = 5


def _emb_lookup(idx, table, n_per, chunk, scale):
    N, = idx.shape
    V, D = table.shape
    n_chunks = n_per // chunk
    nfly = _NBUF - 1
    assert n_chunks % _NBUF == 0 and n_chunks >= 2 * _NBUF
    nc = 2
    mesh = plsc.VectorSubcoreMesh(core_axis_name="c", subcore_axis_name="s")

    @functools.partial(
        pl.kernel,
        mesh=mesh,
        out_type=jax.ShapeDtypeStruct((N, D), jnp.float32),
        scratch_types=[
            pltpu.VMEM((n_per,), jnp.int32),
            [pltpu.VMEM((chunk, D), jnp.float32) for _ in range(_NBUF)],
            pltpu.SemaphoreType.DMA((_NBUF,)),
            pltpu.SemaphoreType.DMA((_NBUF,)),
        ],
        compiler_params=pltpu.CompilerParams(use_tc_tiling_on_sc=False),
    )
    def body(idx_hbm, table_hbm, out_hbm, idx_v, row_bufs, gsem, ssem):
        wid = lax.axis_index("s") * nc + lax.axis_index("c")
        base = wid * n_per

        def start_gather(g, b):
            pltpu.async_copy(table_hbm.at[idx_v.at[pl.ds(g * chunk, chunk)]],
                             row_bufs[b], gsem.at[b])

        def wait_gather(g, b):
            pltpu.make_async_copy(table_hbm.at[idx_v.at[pl.ds(g * chunk, chunk)]],
                                  row_bufs[b], gsem.at[b]).wait()

        def scale_rows(b):
            def grp_body(r8, carry):
                for r in range(8):
                    for j in range(D // 16):
                        sl = pl.ds(j * 16, 16)
                        row_bufs[b][r8 * 8 + r, sl] = (
                            row_bufs[b][r8 * 8 + r, sl] * scale)
                return carry

            lax.fori_loop(0, chunk // 8, grp_body, 0)

        def start_scatter(g, b):
            start = base + g * chunk
            pltpu.async_copy(row_bufs[b], out_hbm.at[pl.ds(start, chunk)],
                             ssem.at[b])

        def wait_scatter(g, b):
            start = base + g * chunk
            pltpu.make_async_copy(row_bufs[b],
                                  out_hbm.at[pl.ds(start, chunk)],
                                  ssem.at[b]).wait()

        pltpu.sync_copy(idx_hbm.at[pl.ds(base, n_per)], idx_v)
        for b in range(nfly):
            start_gather(b, b)

        def grp_body(p, carry):
            for b in range(_NBUF):
                g = p * _NBUF + b
                wait_gather(g, b)
                scale_rows(b)
                start_scatter(g, b)
                bprev = (b + _NBUF - 1) % _NBUF

                @pl.when(g + nfly < n_chunks)
                def _():
                    @pl.when(g >= 1)
                    def _():
                        wait_scatter(g - 1, bprev)

                    start_gather(g + nfly, bprev)

            return carry

        lax.fori_loop(0, n_chunks // _NBUF, grp_body, 0)
        for g in range(n_chunks - _NBUF, n_chunks):
            wait_scatter(g, g % _NBUF)

    return body(idx, table)


def kernel(tokens, embedding):
    B, S = tokens.shape
    V, D = embedding.shape
    N = B * S
    idx = tokens.reshape(N).astype(jnp.int32)
    n_workers = 32
    n_per = N // n_workers
    out = _emb_lookup(idx, embedding, n_per, 256, math.sqrt(D))
    return out.reshape(B, S, D)


# speedup vs baseline: 4.2637x; 1.0011x over previous
import functools
import math

import jax
import jax.numpy as jnp
from jax import lax
from jax.experimental import pallas as pl
from jax.experimental.pallas import tpu as pltpu
from jax.experimental.pallas import tpu_sc as plsc

_NBUF = 4


def _emb_lookup(idx, table, n_per, chunk, scale):
    N, = idx.shape
    V, D = table.shape
    n_chunks = n_per // chunk
    nfly = _NBUF - 1
    assert n_chunks % _NBUF == 0 and n_chunks >= 2 * _NBUF
    nc = 2
    mesh = plsc.VectorSubcoreMesh(core_axis_name="c", subcore_axis_name="s")

    @functools.partial(
        pl.kernel,
        mesh=mesh,
        out_type=jax.ShapeDtypeStruct((N, D), jnp.float32),
        scratch_types=[
            pltpu.VMEM((n_per,), jnp.int32),
            [pltpu.VMEM((chunk, D), jnp.float32) for _ in range(_NBUF)],
            pltpu.SemaphoreType.DMA((_NBUF,)),
            pltpu.SemaphoreType.DMA((_NBUF,)),
        ],
        compiler_params=pltpu.CompilerParams(use_tc_tiling_on_sc=False),
    )
    def body(idx_hbm, table_hbm, out_hbm, idx_v, row_bufs, gsem, ssem):
        wid = lax.axis_index("s") * nc + lax.axis_index("c")
        base = wid * n_per

        def start_gather(g, b):
            pltpu.async_copy(table_hbm.at[idx_v.at[pl.ds(g * chunk, chunk)]],
                             row_bufs[b], gsem.at[b])

        def wait_gather(g, b):
            pltpu.make_async_copy(table_hbm.at[idx_v.at[pl.ds(g * chunk, chunk)]],
                                  row_bufs[b], gsem.at[b]).wait()

        def scale_rows(b):
            def grp_body(r8, carry):
                for r in range(8):
                    for j in range(D // 16):
                        sl = pl.ds(j * 16, 16)
                        row_bufs[b][r8 * 8 + r, sl] = (
                            row_bufs[b][r8 * 8 + r, sl] * scale)
                return carry

            lax.fori_loop(0, chunk // 8, grp_body, 0)

        def start_scatter(g, b):
            start = base + g * chunk
            pltpu.async_copy(row_bufs[b], out_hbm.at[pl.ds(start, chunk)],
                             ssem.at[b])

        def wait_scatter(g, b):
            start = base + g * chunk
            pltpu.make_async_copy(row_bufs[b],
                                  out_hbm.at[pl.ds(start, chunk)],
                                  ssem.at[b]).wait()

        pltpu.sync_copy(idx_hbm.at[pl.ds(base, n_per)], idx_v)
        for b in range(nfly):
            start_gather(b, b)

        def grp_body(p, carry):
            for b in range(_NBUF):
                g = p * _NBUF + b
                bprev = (b + _NBUF - 1) % _NBUF

                @pl.when(g + nfly < n_chunks)
                def _():
                    @pl.when(g >= 1)
                    def _():
                        wait_scatter(g - 1, bprev)

                    start_gather(g + nfly, bprev)

                wait_gather(g, b)
                scale_rows(b)
                start_scatter(g, b)

            return carry

        lax.fori_loop(0, n_chunks // _NBUF, grp_body, 0)
        for g in range(n_chunks - _NBUF, n_chunks):
            wait_scatter(g, g % _NBUF)

    return body(idx, table)


def kernel(tokens, embedding):
    B, S = tokens.shape
    V, D = embedding.shape
    N = B * S
    idx = tokens.reshape(N).astype(jnp.int32)
    n_workers = 32
    n_per = N // n_workers
    out = _emb_lookup(idx, embedding, n_per, 128, math.sqrt(D))
    return out.reshape(B, S, D)


# speedup vs baseline: 4.2680x; 1.0010x over previous
import functools
import math

import jax
import jax.numpy as jnp
from jax import lax
from jax.experimental import pallas as pl
from jax.experimental.pallas import tpu as pltpu
from jax.experimental.pallas import tpu_sc as plsc

_NBUF = 4


def _emb_lookup(idx, table, n_per, chunk, scale):
    N, = idx.shape
    V, D = table.shape
    n_chunks = n_per // chunk
    nfly = _NBUF - 1
    assert n_chunks % _NBUF == 0 and n_chunks >= 2 * _NBUF
    nc = plsc.get_sparse_core_info().num_cores
    mesh = plsc.VectorSubcoreMesh(core_axis_name="c", subcore_axis_name="s")

    @functools.partial(
        pl.kernel,
        mesh=mesh,
        out_type=jax.ShapeDtypeStruct((N, D), jnp.float32),
        scratch_types=[
            pltpu.VMEM((n_per,), jnp.int32),
            [pltpu.VMEM((chunk, D), jnp.float32) for _ in range(_NBUF)],
            pltpu.SemaphoreType.DMA((_NBUF,)),
            pltpu.SemaphoreType.DMA((_NBUF,)),
        ],
        compiler_params=pltpu.CompilerParams(use_tc_tiling_on_sc=False),
    )
    def body(idx_hbm, table_hbm, out_hbm, idx_v, row_bufs, gsem, ssem):
        wid = lax.axis_index("s") * nc + lax.axis_index("c")
        base = wid * n_per

        def start_gather(g, b):
            pltpu.async_copy(table_hbm.at[idx_v.at[pl.ds(g * chunk, chunk)]],
                             row_bufs[b], gsem.at[b])

        def wait_gather(g, b):
            pltpu.make_async_copy(table_hbm.at[idx_v.at[pl.ds(g * chunk, chunk)]],
                                  row_bufs[b], gsem.at[b]).wait()

        def scale_rows(b):
            def grp_body(r8, carry):
                for r in range(8):
                    for j in range(D // 16):
                        sl = pl.ds(j * 16, 16)
                        row_bufs[b][r8 * 8 + r, sl] = (
                            row_bufs[b][r8 * 8 + r, sl] * scale)
                return carry

            lax.fori_loop(0, chunk // 8, grp_body, 0)

        def start_scatter(g, b):
            start = base + g * chunk
            pltpu.async_copy(row_bufs[b], out_hbm.at[pl.ds(start, chunk)],
                             ssem.at[b])

        def wait_scatter(g, b):
            start = base + g * chunk
            pltpu.make_async_copy(row_bufs[b],
                                  out_hbm.at[pl.ds(start, chunk)],
                                  ssem.at[b]).wait()

        pltpu.sync_copy(idx_hbm.at[pl.ds(base, n_per)], idx_v)
        for b in range(nfly):
            start_gather(b, b)

        def grp_body(p, carry):
            for b in range(_NBUF):
                g = p * _NBUF + b
                bprev = (b + _NBUF - 1) % _NBUF

                @pl.when(g + nfly < n_chunks)
                def _():
                    @pl.when(g >= 1)
                    def _():
                        wait_scatter(g - 1, bprev)

                    start_gather(g + nfly, bprev)

                wait_gather(g, b)
                scale_rows(b)
                start_scatter(g, b)

            return carry

        lax.fori_loop(0, n_chunks // _NBUF, grp_body, 0)
        for g in range(n_chunks - _NBUF, n_chunks):
            wait_scatter(g, g % _NBUF)

    return body(idx, table)


def kernel(tokens, embedding):
    B, S = tokens.shape
    V, D = embedding.shape
    N = B * S
    idx = tokens.reshape(N).astype(jnp.int32)
    info = plsc.get_sparse_core_info()
    n_workers = info.num_cores * info.num_subcores
    n_per = N // n_workers
    out = _emb_lookup(idx, embedding, n_per, 128, math.sqrt(D))
    return out.reshape(B, S, D)
